# baseline (speedup 1.0000x reference)
import jax
import jax.numpy as jnp
from jax import lax
from jax.experimental import pallas as pl
from jax.experimental.pallas import tpu as pltpu
from jax.experimental.pallas import tpu_sc as plsc

F32 = jnp.float32
NC = 2
NS = 16
NW = NC * NS


def _mesh():
    return plsc.VectorSubcoreMesh(
        core_axis_name="c", subcore_axis_name="s", num_cores=NC, num_subcores=NS
    )


def _pair_gather(table, idx0, idx1):
    V, D = table.shape
    B = idx0.shape[0]
    per = B // NW
    G = 200
    nch = per // G
    assert per % G == 0 and per % 8 == 0

    def body(table_h, i0_h, i1_h, o0_h, o1_h, idx_v, buf, sem):
        wid = lax.axis_index("s") * NC + lax.axis_index("c")
        base = wid * per
        for idx_h, out_h in ((i0_h, o0_h), (i1_h, o1_h)):
            pltpu.sync_copy(idx_h.at[pl.ds(base, per)], idx_v)

            def step(k, _, out_h=out_h):
                pltpu.async_copy(
                    table_h.at[idx_v.at[pl.ds(k * G, G)]], buf, sem
                ).wait()
                pltpu.sync_copy(buf, out_h.at[pl.ds(base + k * G, G)])
                return 0

            lax.fori_loop(0, nch, step, 0)

    out = jax.ShapeDtypeStruct((B, D), table.dtype)
    return pl.kernel(
        body,
        out_type=(out, out),
        mesh=_mesh(),
        scratch_types=[
            pltpu.VMEM((per,), jnp.int32),
            pltpu.VMEM((G, D), table.dtype),
            pltpu.SemaphoreType.DMA,
        ],
    )(table, idx0, idx1)


def _scatter_add(msg, ids, n_out):
    T, D = msg.shape
    CG = 200
    per = T // NW
    kchunks = per // CG
    zc = ((n_out + NS - 1) // NS + 7) // 8 * 8
    zl = n_out - (NS - 1) * zc
    assert per % CG == 0 and 0 < zl <= zc and zc % 8 == 0 and zl % 8 == 0

    ids2 = ids.reshape(T // CG, CG)
    zeros = jnp.zeros((CG, D), F32)

    def _zplan(rows):
        plan, off = [], 0
        while off < rows:
            sz = min(CG, rows - off)
            plan.append((off, sz))
            off += sz
        return plan

    def body(msg_h, ids_h, zeros_h, o0, o1, ibuf, tbuf, sem):
        c = lax.axis_index("c")
        s = lax.axis_index("s")

        def phase(out_h):
            base = s * zc

            @pl.when(s < NS - 1)
            def _():
                for off, sz in _zplan(zc):
                    pltpu.sync_copy(zeros_h.at[pl.ds(0, sz)],
                                    out_h.at[pl.ds(base + off, sz)])

            @pl.when(s == NS - 1)
            def _():
                for off, sz in _zplan(zl):
                    pltpu.sync_copy(zeros_h.at[pl.ds(0, sz)],
                                    out_h.at[pl.ds(base + off, sz)])

            plsc.subcore_barrier()

            def gstep(k, _):
                g = (c * NS + s) * kchunks + k
                pltpu.sync_copy(ids_h.at[g], ibuf)
                pltpu.async_copy(
                    msg_h.at[pl.ds(g * CG, CG)], tbuf, sem).wait()
                pltpu.sync_copy(tbuf, out_h.at[ibuf], add=True)
                return 0

            lax.fori_loop(0, kchunks, gstep, 0)

        @pl.when(c == 0)
        def _():
            phase(o0)

        @pl.when(c == 1)
        def _():
            phase(o1)

    out = jax.ShapeDtypeStruct((n_out, D), F32)
    return pl.kernel(
        body,
        out_type=(out, out),
        mesh=_mesh(),
        scratch_types=[
            pltpu.VMEM((CG,), jnp.int32),
            pltpu.VMEM((CG, D), F32),
            pltpu.SemaphoreType.DMA,
        ],
        compiler_params=pltpu.CompilerParams(needs_layout_passes=False),
    )(msg, ids2, zeros)


def _tc_msg(rows, xs, wms, bm, wgs, bg):
    T = xs[0].shape[0]
    D = wms[0].shape[1]
    nx = len(xs)
    grid = (T // rows,)

    def body(*refs):
        x = [refs[i][...] for i in range(nx)]
        wm = [refs[nx + i][...] for i in range(nx)]
        bmr = refs[2 * nx][...]
        wg = [refs[2 * nx + 1 + i][...] for i in range(nx)]
        bgr = refs[3 * nx + 1][...]
        out = refs[3 * nx + 2]
        hm = bmr
        hg = bgr
        for xi, wmi, wgi in zip(x, wm, wg):
            hm = hm + jnp.dot(xi, wmi, preferred_element_type=F32)
            hg = hg + jnp.dot(xi, wgi, preferred_element_type=F32)
        out[...] = jax.nn.silu(hm) * jax.nn.sigmoid(hg)

    in_specs = (
        [pl.BlockSpec((rows, x.shape[1]), lambda i: (i, 0)) for x in xs]
        + [pl.BlockSpec(w.shape, lambda i: (0, 0)) for w in wms]
        + [pl.BlockSpec((1, D), lambda i: (0, 0))]
        + [pl.BlockSpec(w.shape, lambda i: (0, 0)) for w in wgs]
        + [pl.BlockSpec((1, D), lambda i: (0, 0))]
    )
    return pl.pallas_call(
        body,
        grid=grid,
        in_specs=in_specs,
        out_specs=pl.BlockSpec((rows, D), lambda i: (i, 0)),
        out_shape=jax.ShapeDtypeStruct((T, D), F32),
    )(*xs, *wms, bm.reshape(1, D), *wgs, bg.reshape(1, D))


def _tc_update(rows, x, a0, a1, w1, w2, b, g, be):
    N, D = x.shape
    grid = (N // rows,)

    def body(x_r, a0_r, a1_r, w1_r, w2_r, b_r, g_r, be_r, out_r):
        xv = x_r[...]
        a = a0_r[...] + a1_r[...]
        h = (jnp.dot(xv, w1_r[...], preferred_element_type=F32)
             + jnp.dot(a, w2_r[...], preferred_element_type=F32)
             + b_r[...])
        mu = jnp.mean(h, axis=-1, keepdims=True)
        var = jnp.mean(h * h, axis=-1, keepdims=True) - mu * mu
        hn = (h - mu) * lax.rsqrt(var + 1e-5) * g_r[...] + be_r[...]
        out_r[...] = xv + jax.nn.silu(hn)

    return pl.pallas_call(
        body,
        grid=grid,
        in_specs=[
            pl.BlockSpec((rows, D), lambda i: (i, 0)),
            pl.BlockSpec((rows, D), lambda i: (i, 0)),
            pl.BlockSpec((rows, D), lambda i: (i, 0)),
            pl.BlockSpec(w1.shape, lambda i: (0, 0)),
            pl.BlockSpec(w2.shape, lambda i: (0, 0)),
            pl.BlockSpec((1, D), lambda i: (0, 0)),
            pl.BlockSpec((1, D), lambda i: (0, 0)),
            pl.BlockSpec((1, D), lambda i: (0, 0)),
        ],
        out_specs=pl.BlockSpec((rows, D), lambda i: (i, 0)),
        out_shape=jax.ShapeDtypeStruct((N, D), F32),
    )(x, a0, a1, w1, w2, b.reshape(1, D), g.reshape(1, D), be.reshape(1, D))


def kernel(atoms, bonds, edge_index, triplets, angle_feat,
           W_bm, b_bm, W_bg, b_bg, W_bu, b_bu, g_bu, be_bu,
           W_am, b_am, W_ag, b_ag, W_au, b_au, g_au, be_au):
    D = bonds.shape[1]
    t0 = triplets[0]
    t1 = triplets[1]
    e0 = edge_index[0]
    e1 = edge_index[1]

    b_ij, b_kj = _pair_gather(bonds, t0, t1)
    msg = _tc_msg(
        640, [b_ij, b_kj, angle_feat],
        [W_bm[:D], W_bm[D:2 * D], W_bm[2 * D:]], b_bm,
        [W_bg[:D], W_bg[D:2 * D], W_bg[2 * D:]], b_bg)
    agg0, agg1 = _scatter_add(msg, t0, bonds.shape[0])
    bonds2 = _tc_update(640, bonds, agg0, agg1,
                        W_bu[:D], W_bu[D:], b_bu, g_bu, be_bu)

    a_src, a_dst = _pair_gather(atoms, e0, e1)
    msg2 = _tc_msg(
        640, [a_src, a_dst, bonds2],
        [W_am[:D], W_am[D:2 * D], W_am[2 * D:]], b_am,
        [W_ag[:D], W_ag[D:2 * D], W_ag[2 * D:]], b_ag)
    agg2a, agg2b = _scatter_add(msg2, e1, atoms.shape[0])
    atoms2 = _tc_update(1000, atoms, agg2a, agg2b,
                        W_au[:D], W_au[D:], b_au, g_au, be_au)

    return atoms2, bonds2
